# baseline (speedup 1.0000x reference)
import functools

import jax
import jax.numpy as jnp
from jax import lax
from jax.experimental import pallas as pl
from jax.experimental.pallas import tpu as pltpu
from jax.experimental.pallas import tpu_sc as plsc

B = 1024
J = 2600
V = 2600
VP = 2601
F = 16
L = 16
NW = 32
ROWS_PER_TILE = B // NW
CHUNK_ROWS = 8
NCHUNK = ROWS_PER_TILE // CHUNK_ROWS
NGROUP = J // L
TAIL = NGROUP * L - (L - (J - NGROUP * L))
UNROLL = 2


def _sc_stage1(x, embT, wp, xbf):
    mesh = plsc.VectorSubcoreMesh(core_axis_name="c", subcore_axis_name="s")

    @functools.partial(
        pl.kernel,
        out_type=(
            jax.ShapeDtypeStruct((B, F * L), jnp.float32),
            jax.ShapeDtypeStruct((B, F * L), jnp.float32),
            jax.ShapeDtypeStruct((B, L), jnp.float32),
        ),
        mesh=mesh,
        compiler_params=pltpu.CompilerParams(needs_layout_passes=False),
        scratch_types=[
            pltpu.VMEM((F * VP,), jnp.float32),
            pltpu.VMEM((J,), jnp.float32),
            pltpu.VMEM((VP + 7,), jnp.float32),
            pltpu.VMEM((CHUNK_ROWS * J,), jnp.int32),
            pltpu.VMEM((CHUNK_ROWS * J,), jnp.int32),
            pltpu.VMEM((ROWS_PER_TILE, F * L), jnp.float32),
            pltpu.VMEM((ROWS_PER_TILE, F * L), jnp.float32),
            pltpu.VMEM((ROWS_PER_TILE, L), jnp.float32),
            pltpu.SemaphoreType.DMA,
            pltpu.SemaphoreType.DMA,
            pltpu.SemaphoreType.DMA,
        ],
    )
    def k(x_hbm, embT_hbm, w_hbm, xbf_hbm, s_hbm, q_hbm, lin_hbm,
          embT_v, w_v, xbf_v, xbuf0, xbuf1, sbuf, qbuf, linbuf,
          sem_t, sem_a, sem_b):
        xbufs = (xbuf0, xbuf1)
        wid = lax.axis_index("s") * 2 + lax.axis_index("c")
        base = wid * ROWS_PER_TILE

        cp_t = pltpu.async_copy(embT_hbm, embT_v, sem_t)
        cp_w = pltpu.async_copy(w_hbm, w_v, sem_t)
        cp_x = pltpu.async_copy(xbf_hbm, xbf_v, sem_t)
        sems = (sem_a, sem_b)
        cps = [None, None]
        cps[0] = pltpu.async_copy(
            x_hbm.at[pl.ds(base * J, CHUNK_ROWS * J)], xbufs[0], sems[0])
        cp_t.wait()
        cp_w.wait()
        cp_x.wait()

        lane = lax.iota(jnp.int32, L)
        tailm = lane >= (L - (J - NGROUP * L))
        padv = jnp.full((L,), V, jnp.int32)
        zerov = jnp.zeros((L,), jnp.float32)

        def body_at(idx, w, carry):
            xf = plsc.load_gather(xbf_v, [idx])
            lin = carry[2 * F] + xf * w
            addr = idx
            acc = list(carry)
            for f in range(F):
                vals = plsc.load_gather(embT_v, [addr])
                acc[f] = acc[f] + vals
                acc[F + f] = acc[F + f] + vals * vals
                if f + 1 < F:
                    addr = addr + VP
            acc[2 * F] = lin
            return tuple(acc)

        def do_row(rr, chunk_buf, rbase):
            init = tuple(zerov for _ in range(2 * F + 1))

            def g_body(g, carry):
                off = pl.multiple_of(rbase + g * L, 8)
                idx = chunk_buf[pl.ds(off, L)]
                w = w_v[pl.ds(pl.multiple_of(g * L, 8), L)]
                return body_at(idx, w, carry)

            mid = plsc.parallel_loop(
                0, NGROUP, unroll=UNROLL, carry=init)(g_body)
            idx_t = chunk_buf[pl.ds(pl.multiple_of(rbase + TAIL, 8), L)]
            idx_t = jnp.where(tailm, idx_t, padv)
            w_t = jnp.where(tailm, w_v[pl.ds(TAIL, L)], zerov)
            fin = body_at(idx_t, w_t, mid)

            for f in range(F):
                sbuf[rr, pl.ds(f * L, L)] = fin[f]
                qbuf[rr, pl.ds(f * L, L)] = fin[F + f]
            linbuf[rr, :] = fin[2 * F]

        for c in range(NCHUNK):
            if c + 1 < NCHUNK:
                cps[(c + 1) % 2] = pltpu.async_copy(
                    x_hbm.at[pl.ds((base + (c + 1) * CHUNK_ROWS) * J,
                                   CHUNK_ROWS * J)],
                    xbufs[(c + 1) % 2], sems[(c + 1) % 2])
            cps[c % 2].wait()

            def row_body(r, _, c=c):
                do_row(c * CHUNK_ROWS + r, xbufs[c % 2], r * J)
                return 0

            lax.fori_loop(0, CHUNK_ROWS, row_body, 0)

        pltpu.sync_copy(sbuf, s_hbm.at[pl.ds(base, ROWS_PER_TILE), :])
        pltpu.sync_copy(qbuf, q_hbm.at[pl.ds(base, ROWS_PER_TILE), :])
        pltpu.sync_copy(linbuf, lin_hbm.at[pl.ds(base, ROWS_PER_TILE), :])

    return k(x, embT, wp, xbf)


def _tc_stage2_body(s_ref, q_ref, lin_ref, b_ref, o_ref):
    sel_r = lax.broadcasted_iota(jnp.int32, (F * L, F), 0) // L
    sel_c = lax.broadcasted_iota(jnp.int32, (F * L, F), 1)
    sel = (sel_r == sel_c).astype(jnp.float32)
    s = jnp.dot(s_ref[...], sel, preferred_element_type=jnp.float32,
                precision=lax.Precision.HIGHEST)
    q = jnp.dot(q_ref[...], sel, preferred_element_type=jnp.float32,
                precision=lax.Precision.HIGHEST)
    s2 = s * s
    m1 = jnp.max(s2)
    m2 = jnp.max(q)
    inter = 0.5 * (jnp.sum(s2, axis=1, keepdims=True) / m1
                   - jnp.sum(q, axis=1, keepdims=True) / m2)
    lin = jnp.sum(lin_ref[...], axis=1, keepdims=True) + b_ref[0, 0]
    o_ref[...] = jax.nn.sigmoid(lin + inter)


def _tc_stage2(s_part, q_part, lin_part, b_lin):
    return pl.pallas_call(
        _tc_stage2_body,
        out_shape=jax.ShapeDtypeStruct((B, 1), jnp.float32),
    )(s_part, q_part, lin_part, b_lin.reshape(1, 1))


def kernel(x, emb, W_lin, b_lin):
    x_flat = x.astype(jnp.int32).reshape(-1)
    emb_pad = jnp.concatenate(
        [emb, jnp.zeros((1, F), jnp.float32)], axis=0)
    embT = emb_pad.T.reshape(-1)
    wp = W_lin.reshape(-1).astype(jnp.bfloat16).astype(jnp.float32)
    xbf = jnp.pad(
        jnp.arange(VP, dtype=jnp.float32).astype(jnp.bfloat16).astype(
            jnp.float32), (0, 7))
    s_part, q_part, lin_part = _sc_stage1(x_flat, embT, wp, xbf)
    out = _tc_stage2(s_part, q_part, lin_part, b_lin)
    return jnp.squeeze(out, axis=1)


# speedup vs baseline: 1.1237x; 1.1237x over previous
import functools

import jax
import jax.numpy as jnp
from jax import lax
from jax.experimental import pallas as pl
from jax.experimental.pallas import tpu as pltpu
from jax.experimental.pallas import tpu_sc as plsc

B = 1024
J = 2600
V = 2600
VP = 2601
F = 16
L = 16
NW = 32
ROWS_PER_TILE = B // NW
CHUNK_ROWS = 8
NCHUNK = ROWS_PER_TILE // CHUNK_ROWS
NGROUP = J // L
TAIL = NGROUP * L - (L - (J - NGROUP * L))
GUNROLL = 2


def _sc_stage1(x, embT, wp, xbf):
    mesh = plsc.VectorSubcoreMesh(core_axis_name="c", subcore_axis_name="s")

    @functools.partial(
        pl.kernel,
        out_type=(
            jax.ShapeDtypeStruct((B, F * L), jnp.float32),
            jax.ShapeDtypeStruct((B, F * L), jnp.float32),
            jax.ShapeDtypeStruct((B, L), jnp.float32),
        ),
        mesh=mesh,
        compiler_params=pltpu.CompilerParams(needs_layout_passes=False),
        scratch_types=[
            pltpu.VMEM((F * VP,), jnp.float32),
            pltpu.VMEM((J,), jnp.float32),
            pltpu.VMEM((VP + 7,), jnp.float32),
            pltpu.VMEM((CHUNK_ROWS, J), jnp.int32),
            pltpu.VMEM((CHUNK_ROWS, J), jnp.int32),
            pltpu.VMEM((ROWS_PER_TILE, F * L), jnp.float32),
            pltpu.VMEM((ROWS_PER_TILE, F * L), jnp.float32),
            pltpu.VMEM((ROWS_PER_TILE, L), jnp.float32),
            pltpu.SemaphoreType.DMA,
            pltpu.SemaphoreType.DMA,
            pltpu.SemaphoreType.DMA,
        ],
    )
    def k(x_hbm, embT_hbm, w_hbm, xbf_hbm, s_hbm, q_hbm, lin_hbm,
          embT_v, w_v, xbf_v, xbuf0, xbuf1, sbuf, qbuf, linbuf,
          sem_t, sem_a, sem_b):
        xbufs = (xbuf0, xbuf1)
        wid = lax.axis_index("s") * 2 + lax.axis_index("c")
        base = wid * ROWS_PER_TILE

        cp_t = pltpu.async_copy(embT_hbm, embT_v, sem_t)
        cp_w = pltpu.async_copy(w_hbm, w_v, sem_t)
        cp_x = pltpu.async_copy(xbf_hbm, xbf_v, sem_t)
        sems = (sem_a, sem_b)
        cps = [None, None]
        cps[0] = pltpu.async_copy(
            x_hbm.at[pl.ds(base, CHUNK_ROWS), :], xbufs[0], sems[0])
        cp_t.wait()
        cp_w.wait()
        cp_x.wait()

        lane = lax.iota(jnp.int32, L)
        tailm = lane >= (L - (J - NGROUP * L))
        padv = jnp.full((L,), V, jnp.int32)
        zerov = jnp.zeros((L,), jnp.float32)

        def body_at(idx, w, carry):
            xf = plsc.load_gather(xbf_v, [idx])
            lin = carry[2 * F] + xf * w
            addr = idx
            acc = list(carry)
            for f in range(F):
                vals = plsc.load_gather(embT_v, [addr])
                acc[f] = acc[f] + vals
                acc[F + f] = acc[F + f] + vals * vals
                if f + 1 < F:
                    addr = addr + VP
            acc[2 * F] = lin
            return tuple(acc)

        def do_row(rr, chunk_buf, r_in_chunk):
            init = tuple(zerov for _ in range(2 * F + 1))

            def g_body(i, carry):
                for u in range(GUNROLL):
                    off = pl.multiple_of((i * GUNROLL + u) * L, 8)
                    idx = chunk_buf[r_in_chunk, pl.ds(off, L)]
                    w = w_v[pl.ds(off, L)]
                    carry = body_at(idx, w, carry)
                return carry

            mid = lax.fori_loop(0, NGROUP // GUNROLL, g_body, init)
            idx_t = chunk_buf[r_in_chunk, pl.ds(TAIL, L)]
            idx_t = jnp.where(tailm, idx_t, padv)
            w_t = jnp.where(tailm, w_v[pl.ds(TAIL, L)], zerov)
            fin = body_at(idx_t, w_t, mid)

            for f in range(F):
                sbuf[rr, pl.ds(f * L, L)] = fin[f]
                qbuf[rr, pl.ds(f * L, L)] = fin[F + f]
            linbuf[rr, :] = fin[2 * F]

        for c in range(NCHUNK):
            if c + 1 < NCHUNK:
                cps[(c + 1) % 2] = pltpu.async_copy(
                    x_hbm.at[pl.ds(base + (c + 1) * CHUNK_ROWS, CHUNK_ROWS), :],
                    xbufs[(c + 1) % 2], sems[(c + 1) % 2])
            cps[c % 2].wait()

            def row_body(r, _, c=c):
                do_row(c * CHUNK_ROWS + r, xbufs[c % 2], r)
                return 0

            lax.fori_loop(0, CHUNK_ROWS, row_body, 0)

        pltpu.sync_copy(sbuf, s_hbm.at[pl.ds(base, ROWS_PER_TILE), :])
        pltpu.sync_copy(qbuf, q_hbm.at[pl.ds(base, ROWS_PER_TILE), :])
        pltpu.sync_copy(linbuf, lin_hbm.at[pl.ds(base, ROWS_PER_TILE), :])

    return k(x, embT, wp, xbf)


def _tc_stage2_body(s_ref, q_ref, lin_ref, b_ref, o_ref):
    sel_r = lax.broadcasted_iota(jnp.int32, (F * L, F), 0) // L
    sel_c = lax.broadcasted_iota(jnp.int32, (F * L, F), 1)
    sel = (sel_r == sel_c).astype(jnp.float32)
    s = jnp.dot(s_ref[...], sel, preferred_element_type=jnp.float32,
                precision=lax.Precision.HIGHEST)
    q = jnp.dot(q_ref[...], sel, preferred_element_type=jnp.float32,
                precision=lax.Precision.HIGHEST)
    s2 = s * s
    m1 = jnp.max(s2)
    m2 = jnp.max(q)
    inter = 0.5 * (jnp.sum(s2, axis=1, keepdims=True) / m1
                   - jnp.sum(q, axis=1, keepdims=True) / m2)
    lin = jnp.sum(lin_ref[...], axis=1, keepdims=True) + b_ref[0, 0]
    o_ref[...] = jax.nn.sigmoid(lin + inter)


def _tc_stage2(s_part, q_part, lin_part, b_lin):
    return pl.pallas_call(
        _tc_stage2_body,
        out_shape=jax.ShapeDtypeStruct((B, 1), jnp.float32),
    )(s_part, q_part, lin_part, b_lin.reshape(1, 1))


def kernel(x, emb, W_lin, b_lin):
    x32 = x.astype(jnp.int32)
    emb_pad = jnp.concatenate(
        [emb, jnp.zeros((1, F), jnp.float32)], axis=0)
    embT = emb_pad.T.reshape(-1)
    wp = W_lin.reshape(-1).astype(jnp.bfloat16).astype(jnp.float32)
    xbf = jnp.pad(
        jnp.arange(VP, dtype=jnp.float32).astype(jnp.bfloat16).astype(
            jnp.float32), (0, 7))
    s_part, q_part, lin_part = _sc_stage1(x32, embT, wp, xbf)
    out = _tc_stage2(s_part, q_part, lin_part, b_lin)
    return jnp.squeeze(out, axis=1)


# speedup vs baseline: 1.1272x; 1.0031x over previous
import functools

import jax
import jax.numpy as jnp
from jax import lax
from jax.experimental import pallas as pl
from jax.experimental.pallas import tpu as pltpu
from jax.experimental.pallas import tpu_sc as plsc

B = 1024
J = 2600
V = 2600
VP = 2601
F = 16
L = 16
NW = 32
ROWS_PER_TILE = B // NW
CHUNK_ROWS = 8
NCHUNK = ROWS_PER_TILE // CHUNK_ROWS
NGROUP = J // L
TAIL = NGROUP * L - (L - (J - NGROUP * L))
GUNROLL = 2


def _sc_stage1(x, embT, wp, xbf):
    mesh = plsc.VectorSubcoreMesh(core_axis_name="c", subcore_axis_name="s")

    @functools.partial(
        pl.kernel,
        out_type=(
            jax.ShapeDtypeStruct((B, F * L), jnp.float32),
            jax.ShapeDtypeStruct((B, F * L), jnp.float32),
            jax.ShapeDtypeStruct((B, L), jnp.float32),
        ),
        mesh=mesh,
        compiler_params=pltpu.CompilerParams(needs_layout_passes=False),
        scratch_types=[
            pltpu.VMEM((F * VP,), jnp.float32),
            pltpu.VMEM((J,), jnp.float32),
            pltpu.VMEM((VP + 7,), jnp.float32),
            pltpu.VMEM((CHUNK_ROWS, J), jnp.int32),
            pltpu.VMEM((CHUNK_ROWS, J), jnp.int32),
            pltpu.VMEM((ROWS_PER_TILE, F * L), jnp.float32),
            pltpu.VMEM((ROWS_PER_TILE, F * L), jnp.float32),
            pltpu.VMEM((ROWS_PER_TILE, L), jnp.float32),
            pltpu.SemaphoreType.DMA,
            pltpu.SemaphoreType.DMA,
            pltpu.SemaphoreType.DMA,
        ],
    )
    def k(x_hbm, embT_hbm, w_hbm, xbf_hbm, s_hbm, q_hbm, lin_hbm,
          embT_v, w_v, xbf_v, xbuf0, xbuf1, sbuf, qbuf, linbuf,
          sem_t, sem_a, sem_b):
        xbufs = (xbuf0, xbuf1)
        wid = lax.axis_index("s") * 2 + lax.axis_index("c")
        base = wid * ROWS_PER_TILE

        cp_t = pltpu.async_copy(embT_hbm, embT_v, sem_t)
        cp_w = pltpu.async_copy(w_hbm, w_v, sem_t)
        cp_x = pltpu.async_copy(xbf_hbm, xbf_v, sem_t)
        sems = (sem_a, sem_b)
        cps = [None, None]
        cps[0] = pltpu.async_copy(
            x_hbm.at[pl.ds(base, CHUNK_ROWS), :], xbufs[0], sems[0])
        cp_t.wait()
        cp_w.wait()
        cp_x.wait()

        lane = lax.iota(jnp.int32, L)
        tailm = lane >= (L - (J - NGROUP * L))
        padv = jnp.full((L,), V, jnp.int32)
        zerov = jnp.zeros((L,), jnp.float32)

        def body_at(idx, w, carry):
            xf = plsc.load_gather(xbf_v, [idx])
            lin = carry[2 * F] + xf * w
            addr = idx
            acc = list(carry)
            for f in range(F):
                vals = plsc.load_gather(embT_v, [addr])
                acc[f] = acc[f] + vals
                acc[F + f] = acc[F + f] + vals * vals
                if f + 1 < F:
                    addr = addr + VP
            acc[2 * F] = lin
            return tuple(acc)

        def do_row(rr, chunk_buf, r_in_chunk):
            init = tuple(zerov for _ in range(2 * F + 1))

            def g_body(i, carry):
                for u in range(GUNROLL):
                    off = pl.multiple_of((i * GUNROLL + u) * L, 8)
                    idx = chunk_buf[r_in_chunk, pl.ds(off, L)]
                    w = w_v[pl.ds(off, L)]
                    carry = body_at(idx, w, carry)
                return carry

            mid = lax.fori_loop(0, NGROUP // GUNROLL, g_body, init)
            idx_t = chunk_buf[r_in_chunk, pl.ds(TAIL, L)]
            idx_t = jnp.where(tailm, idx_t, padv)
            w_t = jnp.where(tailm, w_v[pl.ds(TAIL, L)], zerov)
            fin = body_at(idx_t, w_t, mid)

            for f in range(F):
                sbuf[rr, pl.ds(f * L, L)] = fin[f]
                qbuf[rr, pl.ds(f * L, L)] = fin[F + f]
            linbuf[rr, :] = fin[2 * F]

        for c in range(NCHUNK):
            if c + 1 < NCHUNK:
                cps[(c + 1) % 2] = pltpu.async_copy(
                    x_hbm.at[pl.ds(base + (c + 1) * CHUNK_ROWS, CHUNK_ROWS), :],
                    xbufs[(c + 1) % 2], sems[(c + 1) % 2])
            cps[c % 2].wait()

            def row_body(r, _, c=c):
                do_row(c * CHUNK_ROWS + r, xbufs[c % 2], r)
                return 0

            lax.fori_loop(0, CHUNK_ROWS, row_body, 0)

        pltpu.sync_copy(sbuf, s_hbm.at[pl.ds(base, ROWS_PER_TILE), :])
        pltpu.sync_copy(qbuf, q_hbm.at[pl.ds(base, ROWS_PER_TILE), :])
        pltpu.sync_copy(linbuf, lin_hbm.at[pl.ds(base, ROWS_PER_TILE), :])

    return k(x, embT, wp, xbf)


def _tc_stage2_body(s_ref, q_ref, lin_ref, b_ref, o_ref):
    sel_r = lax.broadcasted_iota(jnp.int32, (F * L, F), 0) // L
    sel_c = lax.broadcasted_iota(jnp.int32, (F * L, F), 1)
    sel = (sel_r == sel_c).astype(jnp.float32)
    s = jnp.dot(s_ref[...], sel, preferred_element_type=jnp.float32,
                precision=lax.Precision.HIGHEST)
    q = jnp.dot(q_ref[...], sel, preferred_element_type=jnp.float32,
                precision=lax.Precision.HIGHEST)
    s2 = s * s
    m1 = jnp.max(s2)
    m2 = jnp.max(q)
    inter = 0.5 * (jnp.sum(s2, axis=1, keepdims=True) / m1
                   - jnp.sum(q, axis=1, keepdims=True) / m2)
    lin = jnp.sum(lin_ref[...], axis=1, keepdims=True) + b_ref[0, 0]
    o_ref[...] = jax.nn.sigmoid(lin + inter)


def _tc_stage2(s_part, q_part, lin_part, b_lin):
    return pl.pallas_call(
        _tc_stage2_body,
        out_shape=jax.ShapeDtypeStruct((B, 1), jnp.float32),
    )(s_part, q_part, lin_part, b_lin.reshape(1, 1))


def _round_bf16(v):
    u = lax.bitcast_convert_type(v, jnp.uint32)
    r = ((u + jnp.uint32(0x7FFF) + ((u >> 16) & jnp.uint32(1)))
         & jnp.uint32(0xFFFF0000))
    return lax.bitcast_convert_type(r, jnp.float32)


def kernel(x, emb, W_lin, b_lin):
    x32 = x.astype(jnp.int32)
    emb_pad = jnp.concatenate(
        [emb, jnp.zeros((1, F), jnp.float32)], axis=0)
    embT = emb_pad.T.reshape(-1)
    wp = _round_bf16(W_lin.reshape(-1))
    xbf = jnp.pad(
        _round_bf16(jnp.arange(VP, dtype=jnp.float32)), (0, 7))
    s_part, q_part, lin_part = _sc_stage1(x32, embT, wp, xbf)
    out = _tc_stage2(s_part, q_part, lin_part, b_lin)
    return jnp.squeeze(out, axis=1)
